# baseline (speedup 1.0000x reference)
import jax
import jax.numpy as jnp
from jax import lax
from jax.experimental import pallas as pl
from jax.experimental.pallas import tpu as pltpu
from jax.experimental.pallas import tpu_sc as plsc

RES = 1024
N_RAYS = 1048576
NC = 2
NS = 16
L = 16
NW = NC * NS
SB = 64


def _face_uv_idx(x, y, z):
    ax = jnp.abs(x)
    ay = jnp.abs(z)
    az = jnp.abs(y)
    ny = -y
    isx = (ax >= ay) & (ax >= az)
    isy = jnp.logical_not(isx) & (ay >= az)
    face = jnp.where(
        isx,
        jnp.where(x >= 0, 0, 1),
        jnp.where(isy,
                  jnp.where(z >= 0, 2, 3),
                  jnp.where(ny >= 0, 4, 5)),
    ).astype(jnp.int32)
    ma = jnp.where(isx, ax, jnp.where(isy, ay, az))
    ma = jnp.maximum(ma, 1e-12)
    sc = jnp.where(
        isx,
        jnp.where(x >= 0, y, ny),
        jnp.where(isy, x, jnp.where(ny >= 0, x, -x)),
    )
    tc = jnp.where(
        isx, -z,
        jnp.where(isy, jnp.where(z >= 0, ny, y), -z),
    )
    u = (sc / ma + 1.0) * 0.5
    v = (tc / ma + 1.0) * 0.5
    fx = u * RES - 0.5
    fy = v * RES - 0.5
    tx = fx.astype(jnp.int32)
    x0 = jnp.where(tx.astype(jnp.float32) > fx, tx - 1, tx)
    ty = fy.astype(jnp.int32)
    y0 = jnp.where(ty.astype(jnp.float32) > fy, ty - 1, ty)
    wx = fx - x0.astype(jnp.float32)
    wy = fy - y0.astype(jnp.float32)
    x0i = jnp.clip(x0, 0, RES - 1)
    x1i = jnp.clip(x0 + 1, 0, RES - 1)
    y0i = jnp.clip(y0, 0, RES - 1)
    y1i = jnp.clip(y0 + 1, 0, RES - 1)
    fb = face << 20
    r0b = fb + (y0i << 10)
    r1b = fb + (y1i << 10)
    return r0b + x0i, r0b + x1i, r1b + x0i, r1b + x1i, wx, wy


def _build(n_rays, chunk, interpret=False):
    rpw = n_rays // NW
    nchunk = rpw // chunk
    ce = 3 * chunk
    ns = chunk // SB

    def body(vd_hbm, table_hbm, out_hbm,
             vv, iT, iB, packv, wxv, wyv,
             cT, cB, outv, sem):
        cid = lax.axis_index("c")
        sid = lax.axis_index("s")
        wid = sid * NC + cid
        base_ray = wid * rpw
        iota = lax.iota(jnp.int32, L)

        def chunk_body(g, carry):
            r0 = base_ray + g * chunk
            pltpu.sync_copy(vd_hbm.at[pl.ds(r0 * 3, 3 * chunk)], vv)

            def comp_body(k, carry2):
                kb = jnp.broadcast_to(k, (L,))
                for jj in range(SB // L):
                    o = k * SB + jj * L
                    rays = iota + o
                    r3 = rays * 3
                    x = plsc.load_gather(vv, [r3])
                    y = plsc.load_gather(vv, [r3 + 1])
                    z = plsc.load_gather(vv, [r3 + 2])
                    i00, i01, i10, i11, wx, wy = _face_uv_idx(x, y, z)
                    pos = (iota + jj * L) * 2
                    plsc.store_scatter(iT, [kb, pos],
                                       lax.shift_right_logical(i00, 1))
                    plsc.store_scatter(iT, [kb, pos + 1],
                                       lax.shift_right_logical(i01, 1))
                    plsc.store_scatter(iB, [kb, pos],
                                       lax.shift_right_logical(i10, 1))
                    plsc.store_scatter(iB, [kb, pos + 1],
                                       lax.shift_right_logical(i11, 1))
                    s = pl.ds(o, L)
                    packv[s] = ((i00 & 1) + ((i01 & 1) << 1)
                                + ((i10 & 1) << 2) + ((i11 & 1) << 3))
                    wxv[s] = wx
                    wyv[s] = wy
                return carry2

            lax.fori_loop(0, ns, comp_body, 0)

            descs = []
            for k in range(ns):
                descs.append(pltpu.async_copy(
                    table_hbm.at[iT.at[k]], cT.at[k], sem))
                descs.append(pltpu.async_copy(
                    table_hbm.at[iB.at[k]], cB.at[k], sem))
            for d in descs:
                d.wait()

            def blend_body(bb, carry2):
                for jj in range(2 * SB // L):
                    e0 = bb * 2 * SB + jj * L
                    e = iota + e0
                    ray = lax.shift_right_logical(e * 21846, 16)
                    k = lax.shift_right_logical(ray, 6)
                    p2 = (ray & (SB - 1)) * 2
                    ch = e - ray * 3
                    wx = plsc.load_gather(wxv, [ray])
                    wy = plsc.load_gather(wyv, [ray])
                    pk = plsc.load_gather(packv, [ray])
                    v00 = plsc.load_gather(cT, [k, p2, ((pk & 1) << 2) + ch])
                    v01 = plsc.load_gather(cT, [k, p2 + 1, ((pk & 2) << 1) + ch])
                    v10 = plsc.load_gather(cB, [k, p2, (pk & 4) + ch])
                    v11 = plsc.load_gather(cB, [k, p2 + 1, ((pk & 8) >> 1) + ch])
                    top = (1.0 - wx) * v00 + wx * v01
                    bot = (1.0 - wx) * v10 + wx * v11
                    outv[pl.ds(e0, L)] = (1.0 - wy) * top + wy * bot
                return carry2

            lax.fori_loop(0, 3 * ns // 2, blend_body, 0)
            pltpu.sync_copy(outv, out_hbm.at[pl.ds(r0 * 3, ce)])
            return carry

        lax.fori_loop(0, nchunk, chunk_body, 0)

    mesh = plsc.VectorSubcoreMesh(
        core_axis_name="c", subcore_axis_name="s",
        num_cores=NC, num_subcores=NS)
    return pl.kernel(
        body,
        out_type=jax.ShapeDtypeStruct((n_rays * 3,), jnp.float32),
        mesh=mesh,
        compiler_params=pltpu.CompilerParams(
            needs_layout_passes=False, use_tc_tiling_on_sc=False),
        scratch_types=[
            pltpu.VMEM((3 * chunk,), jnp.float32),
            pltpu.VMEM((ns, 2 * SB), jnp.int32),
            pltpu.VMEM((ns, 2 * SB), jnp.int32),
            pltpu.VMEM((chunk,), jnp.int32),
            pltpu.VMEM((chunk,), jnp.float32),
            pltpu.VMEM((chunk,), jnp.float32),
            pltpu.VMEM((ns, 2 * SB, 8), jnp.float32),
            pltpu.VMEM((ns, 2 * SB, 8), jnp.float32),
            pltpu.VMEM((ce,), jnp.float32),
            pltpu.SemaphoreType.DMA,
        ],
        interpret=interpret,
    )


@jax.jit
def _run(viewdirs, base):
    n = viewdirs.shape[0]
    fn = _build(n, 1024)
    vd = viewdirs.reshape(-1)
    table = jnp.pad(base, ((0, 0), (0, 0), (0, 0), (0, 1))).reshape(
        6 * RES * RES // 2, 8)
    out = fn(vd, table)
    return out.reshape(n, 3)


def kernel(viewdirs, base):
    return _run(viewdirs, base)


# speedup vs baseline: 1.1355x; 1.1355x over previous
import jax
import jax.numpy as jnp
from jax import lax
from jax.experimental import pallas as pl
from jax.experimental.pallas import tpu as pltpu
from jax.experimental.pallas import tpu_sc as plsc

RES = 1024
N_RAYS = 1048576
NC = 2
NS = 16
L = 16
NW = NC * NS
SB = 64


def _face_uv_idx(x, y, z):
    ax = jnp.abs(x)
    ay = jnp.abs(z)
    az = jnp.abs(y)
    ny = -y
    isx = (ax >= ay) & (ax >= az)
    isy = jnp.logical_not(isx) & (ay >= az)
    face = jnp.where(
        isx,
        jnp.where(x >= 0, 0, 1),
        jnp.where(isy,
                  jnp.where(z >= 0, 2, 3),
                  jnp.where(ny >= 0, 4, 5)),
    ).astype(jnp.int32)
    ma = jnp.where(isx, ax, jnp.where(isy, ay, az))
    ma = jnp.maximum(ma, 1e-12)
    sc = jnp.where(
        isx,
        jnp.where(x >= 0, y, ny),
        jnp.where(isy, x, jnp.where(ny >= 0, x, -x)),
    )
    tc = jnp.where(
        isx, -z,
        jnp.where(isy, jnp.where(z >= 0, ny, y), -z),
    )
    u = (sc / ma + 1.0) * 0.5
    v = (tc / ma + 1.0) * 0.5
    fx = u * RES - 0.5
    fy = v * RES - 0.5
    tx = fx.astype(jnp.int32)
    x0 = jnp.where(tx.astype(jnp.float32) > fx, tx - 1, tx)
    ty = fy.astype(jnp.int32)
    y0 = jnp.where(ty.astype(jnp.float32) > fy, ty - 1, ty)
    wx = fx - x0.astype(jnp.float32)
    wy = fy - y0.astype(jnp.float32)
    x0i = jnp.clip(x0, 0, RES - 1)
    x1i = jnp.clip(x0 + 1, 0, RES - 1)
    y0i = jnp.clip(y0, 0, RES - 1)
    y1i = jnp.clip(y0 + 1, 0, RES - 1)
    fb = face << 20
    r0b = fb + (y0i << 10)
    r1b = fb + (y1i << 10)
    return r0b + x0i, r0b + x1i, r1b + x0i, r1b + x1i, wx, wy


def _build(n_rays, chunk, interpret=False):
    rpw = n_rays // NW
    nchunk = rpw // chunk
    ce = 3 * chunk
    ns = chunk // SB

    def body(vd_hbm, table_hbm, out_hbm,
             vv, iT, iB, oTv, oBv, d01v, wxv, wyv,
             cT, cB, outv, sem):
        cid = lax.axis_index("c")
        sid = lax.axis_index("s")
        wid = sid * NC + cid
        base_ray = wid * rpw
        iota = lax.iota(jnp.int32, L)

        def chunk_body(g, carry):
            r0 = base_ray + g * chunk
            pltpu.sync_copy(vd_hbm.at[pl.ds(r0 * 3, 3 * chunk)], vv)

            def comp_body(k, carry2):
                kb = jnp.broadcast_to(k, (L,))
                for jj in range(SB // L):
                    o = k * SB + jj * L
                    rays = iota + o
                    r3 = rays * 3
                    x = plsc.load_gather(vv, [r3])
                    y = plsc.load_gather(vv, [r3 + 1])
                    z = plsc.load_gather(vv, [r3 + 2])
                    i00, i01, i10, i11, wx, wy = _face_uv_idx(x, y, z)
                    wT = i00 * 3
                    wB = i10 * 3
                    rT = lax.shift_right_logical(wT, 3)
                    rB = lax.shift_right_logical(wB, 3)
                    last = 6 * RES * RES * 3 // 8 - 1
                    pos = (iota + jj * L) * 2
                    plsc.store_scatter(iT, [kb, pos], rT)
                    plsc.store_scatter(iT, [kb, pos + 1],
                                       jnp.minimum(rT + 1, last))
                    plsc.store_scatter(iB, [kb, pos], rB)
                    plsc.store_scatter(iB, [kb, pos + 1],
                                       jnp.minimum(rB + 1, last))
                    s = pl.ds(o, L)
                    oTv[s] = wT & 7
                    oBv[s] = wB & 7
                    d01v[s] = (i01 - i00) * 3
                    wxv[s] = wx
                    wyv[s] = wy
                return carry2

            lax.fori_loop(0, ns, comp_body, 0)

            descs = []
            for k in range(ns):
                descs.append(pltpu.async_copy(
                    table_hbm.at[iT.at[k]], cT.at[k], sem))
                descs.append(pltpu.async_copy(
                    table_hbm.at[iB.at[k]], cB.at[k], sem))
            for d in descs:
                d.wait()

            def blend_body(bb, carry2):
                for jj in range(2 * SB // L):
                    e0 = bb * 2 * SB + jj * L
                    e = iota + e0
                    ray = lax.shift_right_logical(e * 21846, 16)
                    k = lax.shift_right_logical(ray, 6)
                    p2 = (ray & (SB - 1)) * 2
                    ch = e - ray * 3
                    wx = plsc.load_gather(wxv, [ray])
                    wy = plsc.load_gather(wyv, [ray])
                    oT = plsc.load_gather(oTv, [ray]) + ch
                    oB = plsc.load_gather(oBv, [ray]) + ch
                    d01 = plsc.load_gather(d01v, [ray])
                    o01 = oT + d01
                    o11 = oB + d01
                    v00 = plsc.load_gather(
                        cT, [k, p2 + lax.shift_right_logical(oT, 3), oT & 7])
                    v01 = plsc.load_gather(
                        cT, [k, p2 + lax.shift_right_logical(o01, 3), o01 & 7])
                    v10 = plsc.load_gather(
                        cB, [k, p2 + lax.shift_right_logical(oB, 3), oB & 7])
                    v11 = plsc.load_gather(
                        cB, [k, p2 + lax.shift_right_logical(o11, 3), o11 & 7])
                    top = (1.0 - wx) * v00 + wx * v01
                    bot = (1.0 - wx) * v10 + wx * v11
                    outv[pl.ds(e0, L)] = (1.0 - wy) * top + wy * bot
                return carry2

            lax.fori_loop(0, 3 * ns // 2, blend_body, 0)
            pltpu.sync_copy(outv, out_hbm.at[pl.ds(r0 * 3, ce)])
            return carry

        lax.fori_loop(0, nchunk, chunk_body, 0)

    mesh = plsc.VectorSubcoreMesh(
        core_axis_name="c", subcore_axis_name="s",
        num_cores=NC, num_subcores=NS)
    return pl.kernel(
        body,
        out_type=jax.ShapeDtypeStruct((n_rays * 3,), jnp.float32),
        mesh=mesh,
        compiler_params=pltpu.CompilerParams(
            needs_layout_passes=False, use_tc_tiling_on_sc=False),
        scratch_types=[
            pltpu.VMEM((3 * chunk,), jnp.float32),
            pltpu.VMEM((ns, 2 * SB), jnp.int32),
            pltpu.VMEM((ns, 2 * SB), jnp.int32),
            pltpu.VMEM((chunk,), jnp.int32),
            pltpu.VMEM((chunk,), jnp.int32),
            pltpu.VMEM((chunk,), jnp.int32),
            pltpu.VMEM((chunk,), jnp.float32),
            pltpu.VMEM((chunk,), jnp.float32),
            pltpu.VMEM((ns, 2 * SB, 8), jnp.float32),
            pltpu.VMEM((ns, 2 * SB, 8), jnp.float32),
            pltpu.VMEM((ce,), jnp.float32),
            pltpu.SemaphoreType.DMA,
        ],
        interpret=interpret,
    )


@jax.jit
def _run(viewdirs, base):
    n = viewdirs.shape[0]
    fn = _build(n, 1024)
    vd = viewdirs.reshape(-1)
    table = base.reshape(6 * RES * RES * 3 // 8, 8)
    out = fn(vd, table)
    return out.reshape(n, 3)


def kernel(viewdirs, base):
    return _run(viewdirs, base)
